# Initial kernel scaffold; baseline (speedup 1.0000x reference)
import jax
import jax.numpy as jnp
from jax.experimental import pallas as pl


def kernel(node_type, velocity, cells, mesh_pos, is_trainning, params):
    raise NotImplementedError("write your pallas kernel here")


# speedup vs baseline: 2.3946x; 2.3946x over previous
import functools

import jax
import jax.numpy as jnp
from jax import lax
from jax.experimental import pallas as pl
from jax.experimental.pallas import tpu as pltpu
from jax.experimental.pallas import tpu_sc as plsc

F32 = jnp.float32
LAT = 64
NC = 2
NS = 16
NW = NC * NS
CH = 128

_SC_PARAMS = pltpu.CompilerParams(use_tc_tiling_on_sc=False)


def _mm(x, w):
    return lax.dot_general(x, w, (((1,), (0,)), ((), ())),
                           preferred_element_type=F32)


def _ln(h, g, b):
    m = jnp.mean(h, axis=-1, keepdims=True)
    v = jnp.mean((h - m) * (h - m), axis=-1, keepdims=True)
    return (h - m) * lax.rsqrt(v + 1e-5) * g + b


@functools.lru_cache(maxsize=None)
def _make_gather2(pw, chunks, h_pad):
    mesh = plsc.VectorSubcoreMesh(core_axis_name="c", subcore_axis_name="s",
                                  num_cores=NC, num_subcores=NS)

    def body(tab, iu, iv, ou, ov, iu_v, iv_v, bufu, bufv, semu, semv):
        wid = lax.axis_index("s") * NC + lax.axis_index("c")
        pltpu.sync_copy(iu.at[wid], iu_v)
        pltpu.sync_copy(iv.at[wid], iv_v)
        base = wid * pw

        def step(j, carry):
            off = j * CH
            cu = pltpu.async_copy(tab.at[iu_v.at[pl.ds(off, CH)]], bufu, semu)
            cv = pltpu.async_copy(tab.at[iv_v.at[pl.ds(off, CH)]], bufv, semv)
            cu.wait()
            cv.wait()
            pltpu.sync_copy(bufu, ou.at[pl.ds(base + off, CH)])
            pltpu.sync_copy(bufv, ov.at[pl.ds(base + off, CH)])
            return carry

        lax.fori_loop(0, chunks, step, 0)

    return pl.kernel(
        body,
        out_type=[jax.ShapeDtypeStruct((h_pad, 2 * LAT), F32)] * 2,
        mesh=mesh,
        compiler_params=_SC_PARAMS,
        scratch_types=[
            pltpu.VMEM((pw,), jnp.int32),
            pltpu.VMEM((pw,), jnp.int32),
            pltpu.VMEM((CH, 2 * LAT), F32),
            pltpu.VMEM((CH, 2 * LAT), F32),
            pltpu.SemaphoreType.DMA,
            pltpu.SemaphoreType.DMA,
        ],
    )


@functools.lru_cache(maxsize=None)
def _make_scatter_add(n_sc, pw, chunks):
    mesh = plsc.VectorSubcoreMesh(core_axis_name="c", subcore_axis_name="s",
                                  num_cores=NC, num_subcores=NS)
    rs = n_sc // NS
    zb_rows = 128
    assert rs % zb_rows == 0

    def body(vals1, vals2, ridx1, ridx2, agg, idx1_v, idx2_v, idxc, buf, zb,
             spmem):
        cid = lax.axis_index("c")
        sid = lax.axis_index("s")
        wid = sid * NC + cid

        def zrow(i, c):
            def zcol(k, c2):
                zb[i, pl.ds(k * 16, 16)] = jnp.zeros((16,), F32)
                return c2
            lax.fori_loop(0, LAT // 16, zcol, c)
            return c
        lax.fori_loop(0, zb_rows, zrow, 0)

        def zs(k, c):
            pltpu.sync_copy(zb, spmem.at[pl.ds(sid * rs + k * zb_rows, zb_rows)])
            return c
        lax.fori_loop(0, rs // zb_rows, zs, 0)
        plsc.subcore_barrier()

        base = wid * pw
        pltpu.sync_copy(ridx1.at[wid], idx1_v)
        pltpu.sync_copy(ridx2.at[wid], idx2_v)

        def stage_idx(src, j):
            def cp(k, c):
                idxc[pl.ds(k * 16, 16)] = src[pl.ds(j * CH + k * 16, 16)]
                return c
            lax.fori_loop(0, CH // 16, cp, 0)

        def step1(j, c):
            stage_idx(idx1_v, j)
            pltpu.sync_copy(vals1.at[pl.ds(base + j * CH, CH)], buf)
            pltpu.sync_copy(buf, spmem.at[idxc], add=True)
            return c
        lax.fori_loop(0, chunks, step1, 0)

        def step2(j, c):
            stage_idx(idx2_v, j)
            pltpu.sync_copy(vals2.at[pl.ds(base + j * CH, CH)], buf)
            pltpu.sync_copy(buf, spmem.at[idxc], add=True)
            return c
        lax.fori_loop(0, chunks, step2, 0)
        plsc.subcore_barrier()

        def wout(k, c):
            off = sid * rs + k * zb_rows
            pltpu.sync_copy(spmem.at[pl.ds(off, zb_rows)],
                            agg.at[pl.ds(cid * n_sc + off, zb_rows)])
            return c
        lax.fori_loop(0, rs // zb_rows, wout, 0)

    return pl.kernel(
        body,
        out_type=jax.ShapeDtypeStruct((NC * n_sc, LAT), F32),
        mesh=mesh,
        compiler_params=_SC_PARAMS,
        scratch_types=[
            pltpu.VMEM((pw,), jnp.int32),
            pltpu.VMEM((pw,), jnp.int32),
            pltpu.VMEM((CH,), jnp.int32),
            pltpu.VMEM((CH, LAT), F32),
            pltpu.VMEM((zb_rows, LAT), F32),
            pltpu.VMEM_SHARED((n_sc, LAT), F32),
        ],
    )


def _row_spec(rb, w):
    return pl.BlockSpec((rb, w), lambda i: (i, 0))


def _full_spec(shape):
    return pl.BlockSpec(shape, lambda i: tuple(0 for _ in shape))


def _edge_step_body(hu, hv, ef1, ef2, wc, b1, w2, b2, w3, b3, lg, lb,
                    e1n, e1o, e2n, e2o):
    u = hu[...]
    v = hv[...]

    def tail(h0, ef):
        h = jnp.maximum(h0, 0.0)
        h = jnp.maximum(_mm(h, w2[...]) + b2[...], 0.0)
        h = _mm(h, w3[...]) + b3[...]
        y = _ln(h, lg[...], lb[...])
        return y, ef + y

    e1 = ef1[...]
    e2 = ef2[...]
    y1, o1 = tail(u[:, :LAT] + v[:, LAT:] + _mm(e1, wc[...]) + b1[...], e1)
    y2, o2 = tail(v[:, :LAT] + u[:, LAT:] + _mm(e2, wc[...]) + b1[...], e2)
    e1n[...] = y1
    e1o[...] = o1
    e2n[...] = y2
    e2o[...] = o2


@functools.lru_cache(maxsize=None)
def _make_edge_step(h_pad, rb):
    grid = h_pad // rb
    row = _row_spec(rb, LAT)
    wide = _row_spec(rb, 2 * LAT)
    mat = _full_spec((LAT, LAT))
    vec = _full_spec((1, LAT))
    return pl.pallas_call(
        _edge_step_body,
        grid=(grid,),
        in_specs=[wide, wide, row, row, mat, vec, mat, vec, mat, vec, vec,
                  vec],
        out_specs=[row, row, row, row],
        out_shape=[jax.ShapeDtypeStruct((h_pad, LAT), F32)] * 4,
    )


def _node_step_body(nf, a0, a1, wa, wb, b1, w2, b2, w3, b3, lg, lb,
                    wan, wbn, nfo, to):
    agg = a0[...] + a1[...]
    h = _mm(nf[...], wa[...]) + _mm(agg, wb[...]) + b1[...]
    h = jnp.maximum(h, 0.0)
    h = jnp.maximum(_mm(h, w2[...]) + b2[...], 0.0)
    h = _mm(h, w3[...]) + b3[...]
    y = nf[...] + _ln(h, lg[...], lb[...])
    nfo[...] = y
    to[...] = jnp.concatenate([_mm(y, wan[...]), _mm(y, wbn[...])], axis=1)


@functools.lru_cache(maxsize=None)
def _make_node_step(n, rb):
    grid = n // rb
    row = _row_spec(rb, LAT)
    mat = _full_spec((LAT, LAT))
    vec = _full_spec((1, LAT))
    return pl.pallas_call(
        _node_step_body,
        grid=(grid,),
        in_specs=[row, row, row, mat, mat, vec, mat, vec, mat, vec, vec, vec,
                  mat, mat],
        out_specs=[row, _row_spec(rb, 2 * LAT)],
        out_shape=[jax.ShapeDtypeStruct((n, LAT), F32),
                   jax.ShapeDtypeStruct((n, 2 * LAT), F32)],
    )


def _node_feat(nt, vel):
    rb = vel.shape[0]
    col = lax.broadcasted_iota(jnp.int32, (rb, 16), 1)
    f = jnp.where(col == 0, vel[:, 0:1], 0.0)
    f = f + jnp.where(col == 1, vel[:, 1:2], 0.0)
    f = f + ((col - 2) == nt).astype(F32)
    return f


def _edge_feat(pu, pv, sign):
    rb = pu.shape[0]
    col = lax.broadcasted_iota(jnp.int32, (rb, 16), 1)
    rx = (pu[:, 0:1] - pv[:, 0:1]) * sign
    ry = (pu[:, 1:2] - pv[:, 1:2]) * sign
    nrm = jnp.sqrt(rx * rx + ry * ry)
    f = jnp.where(col == 0, rx, 0.0)
    f = f + jnp.where(col == 1, ry, 0.0)
    f = f + jnp.where(col == 2, nrm, 0.0)
    return f


@functools.lru_cache(maxsize=None)
def _make_node_stats(n, rb):
    grid = n // rb

    def body(nt, vel, mo, ro, ssum, ssq):
        i = pl.program_id(0)

        @pl.when(i == 0)
        def _():
            ssum[...] = jnp.zeros_like(ssum)
            ssq[...] = jnp.zeros_like(ssq)

        feat = _node_feat(nt[...], vel[...])
        ssum[...] += jnp.sum(feat, axis=0, keepdims=True)
        ssq[...] += jnp.sum(feat * feat, axis=0, keepdims=True)

        @pl.when(i == grid - 1)
        def _():
            mean = ssum[...] / float(n)
            var = jnp.maximum(ssq[...] / float(n) - mean * mean, 0.0)
            std = jnp.maximum(jnp.sqrt(var), 1e-8)
            mo[...] = mean
            ro[...] = 1.0 / std

    return pl.pallas_call(
        body,
        grid=(grid,),
        in_specs=[_row_spec(rb, 1), _row_spec(rb, 2)],
        out_specs=[_full_spec((1, 16))] * 2,
        out_shape=[jax.ShapeDtypeStruct((1, 16), F32)] * 2,
        scratch_shapes=[pltpu.VMEM((1, 16), F32), pltpu.VMEM((1, 16), F32)],
    )


@functools.lru_cache(maxsize=None)
def _make_edge_stats(h_pad, h, rb):
    grid = h_pad // rb

    def body(pu, pv, mo, ro, ssum, ssq):
        i = pl.program_id(0)

        @pl.when(i == 0)
        def _():
            ssum[...] = jnp.zeros_like(ssum)
            ssq[...] = jnp.zeros_like(ssq)

        f1 = _edge_feat(pu[...], pv[...], 1.0)
        f2 = _edge_feat(pu[...], pv[...], -1.0)
        rowid = i * rb + lax.broadcasted_iota(jnp.int32, (rb, 16), 0)
        valid = rowid < h
        f1 = jnp.where(valid, f1, 0.0)
        f2 = jnp.where(valid, f2, 0.0)
        ssum[...] += jnp.sum(f1 + f2, axis=0, keepdims=True)
        ssq[...] += jnp.sum(f1 * f1 + f2 * f2, axis=0, keepdims=True)

        @pl.when(i == grid - 1)
        def _():
            cnt = float(2 * h)
            mean = ssum[...] / cnt
            var = jnp.maximum(ssq[...] / cnt - mean * mean, 0.0)
            std = jnp.maximum(jnp.sqrt(var), 1e-8)
            mo[...] = mean
            ro[...] = 1.0 / std

    return pl.pallas_call(
        body,
        grid=(grid,),
        in_specs=[_row_spec(rb, 2 * LAT), _row_spec(rb, 2 * LAT)],
        out_specs=[_full_spec((1, 16))] * 2,
        out_shape=[jax.ShapeDtypeStruct((1, 16), F32)] * 2,
        scratch_shapes=[pltpu.VMEM((1, 16), F32), pltpu.VMEM((1, 16), F32)],
    )


def _enc_tail(x, w1, b1, w2, b2, w3, b3, lg, lb):
    h = jnp.maximum(_mm(x, w1[...]) + b1[...], 0.0)
    h = jnp.maximum(_mm(h, w2[...]) + b2[...], 0.0)
    h = _mm(h, w3[...]) + b3[...]
    return _ln(h, lg[...], lb[...])


@functools.lru_cache(maxsize=None)
def _make_node_enc(n, rb):
    grid = n // rb

    def body(nt, vel, mean, rstd, w1, b1, w2, b2, w3, b3, lg, lb, wa, wb,
             nfo, to):
        x = (_node_feat(nt[...], vel[...]) - mean[...]) * rstd[...]
        y = _enc_tail(x, w1, b1, w2, b2, w3, b3, lg, lb)
        nfo[...] = y
        to[...] = jnp.concatenate([_mm(y, wa[...]), _mm(y, wb[...])], axis=1)

    mat = _full_spec((LAT, LAT))
    vec = _full_spec((1, LAT))
    return pl.pallas_call(
        body,
        grid=(grid,),
        in_specs=[_row_spec(rb, 1), _row_spec(rb, 2),
                  _full_spec((1, 16)), _full_spec((1, 16)),
                  _full_spec((16, LAT)), vec, mat, vec, mat, vec, vec, vec,
                  mat, mat],
        out_specs=[_row_spec(rb, LAT), _row_spec(rb, 2 * LAT)],
        out_shape=[jax.ShapeDtypeStruct((n, LAT), F32),
                   jax.ShapeDtypeStruct((n, 2 * LAT), F32)],
    )


@functools.lru_cache(maxsize=None)
def _make_edge_enc(h_pad, rb):
    grid = h_pad // rb

    def body(pu, pv, mean, rstd, w1, b1, w2, b2, w3, b3, lg, lb, e1o, e2o):
        x1 = (_edge_feat(pu[...], pv[...], 1.0) - mean[...]) * rstd[...]
        e1o[...] = _enc_tail(x1, w1, b1, w2, b2, w3, b3, lg, lb)
        x2 = (_edge_feat(pu[...], pv[...], -1.0) - mean[...]) * rstd[...]
        e2o[...] = _enc_tail(x2, w1, b1, w2, b2, w3, b3, lg, lb)

    mat = _full_spec((LAT, LAT))
    vec = _full_spec((1, LAT))
    return pl.pallas_call(
        body,
        grid=(grid,),
        in_specs=[_row_spec(rb, 2 * LAT), _row_spec(rb, 2 * LAT),
                  _full_spec((1, 16)), _full_spec((1, 16)),
                  _full_spec((16, LAT)), vec, mat, vec, mat, vec, vec, vec],
        out_specs=[_row_spec(rb, LAT)] * 2,
        out_shape=[jax.ShapeDtypeStruct((h_pad, LAT), F32)] * 2,
    )


@functools.lru_cache(maxsize=None)
def _make_decode(n, rb):
    grid = n // rb

    def body(nf, w1, b1, w2, b2, w3, b3, out):
        h = jnp.maximum(_mm(nf[...], w1[...]) + b1[...], 0.0)
        h = jnp.maximum(_mm(h, w2[...]) + b2[...], 0.0)
        out[...] = _mm(h, w3[...]) + b3[...]

    mat = _full_spec((LAT, LAT))
    vec = _full_spec((1, LAT))
    return pl.pallas_call(
        body,
        grid=(grid,),
        in_specs=[_row_spec(rb, LAT), mat, vec, mat, vec, mat, vec],
        out_specs=_row_spec(rb, LAT),
        out_shape=jax.ShapeDtypeStruct((n, LAT), F32),
    )


def _vec(b):
    return b.reshape(1, -1).astype(F32)


def kernel(node_type, velocity, cells, mesh_pos, is_trainning, params):
    n = velocity.shape[0]
    c = cells.shape[0]
    h = 3 * c

    cells = cells.astype(jnp.int32)
    c0, c1, c2 = cells[:, 0], cells[:, 1], cells[:, 2]
    su = jnp.concatenate([c0, c1, c2])
    rv = jnp.concatenate([c1, c2, c0])

    chunks = -(-h // (NW * CH))
    pw = chunks * CH
    h_pad = NW * pw
    pad = h_pad - h

    n_sc = -(-n // (NS * 128)) * (NS * 128)
    if n_sc == n:
        n_sc += NS * 128
    dummy = n_sc - 1

    zpad = jnp.zeros((pad,), jnp.int32)
    dpad = jnp.full((pad,), dummy, jnp.int32)
    su_idx = jnp.concatenate([su, zpad]).reshape(NW, pw)
    rv_idx = jnp.concatenate([rv, zpad]).reshape(NW, pw)
    r1_idx = jnp.concatenate([rv, dpad]).reshape(NW, pw)
    r2_idx = jnp.concatenate([su, dpad]).reshape(NW, pw)

    mp128 = jnp.pad(mesh_pos.astype(F32), ((0, 0), (0, 2 * LAT - 2)))

    gather2 = _make_gather2(pw, chunks, h_pad)
    scatter = _make_scatter_add(n_sc, pw, chunks)

    rb_e = 512
    rb_n = 400
    assert h_pad % rb_e == 0 and n % rb_n == 0

    edge_stats = _make_edge_stats(h_pad, h, rb_e)
    edge_enc = _make_edge_enc(h_pad, rb_e)
    node_stats = _make_node_stats(n, rb_n)
    node_enc = _make_node_enc(n, rb_n)
    edge_step = _make_edge_step(h_pad, rb_e)
    node_step = _make_node_step(n, rb_n)
    decode = _make_decode(n, rb_n)

    ne = params["node_enc"]
    ee = params["edge_enc"]
    blocks = params["blocks"]
    dec = params["dec"]["mlp"]

    def enc_args(p, in_w):
        (w1, b1), (w2, b2), (w3, b3) = p["mlp"]
        g, b = p["ln"]
        w1p = jnp.pad(w1.astype(F32), ((0, 16 - in_w), (0, 0)))
        return (w1p, _vec(b1), w2.astype(F32), _vec(b2), w3.astype(F32),
                _vec(b3), _vec(g), _vec(b))

    ew = []
    nw = []
    for blk in blocks:
        (w1, b1), (w2, b2), (w3, b3) = blk["edge"]["mlp"]
        g, b = blk["edge"]["ln"]
        ew.append((w1[:LAT].astype(F32), w1[LAT:2 * LAT].astype(F32),
                   w1[2 * LAT:].astype(F32), _vec(b1), w2.astype(F32),
                   _vec(b2), w3.astype(F32), _vec(b3), _vec(g), _vec(b)))
        (w1, b1), (w2, b2), (w3, b3) = blk["node"]["mlp"]
        g, b = blk["node"]["ln"]
        nw.append((w1[:LAT].astype(F32), w1[LAT:].astype(F32), _vec(b1),
                   w2.astype(F32), _vec(b2), w3.astype(F32), _vec(b3),
                   _vec(g), _vec(b)))

    steps = len(blocks)

    pu, pv = gather2(mp128, su_idx, rv_idx)
    emean, erstd = edge_stats(pu, pv)
    ef1, ef2 = edge_enc(pu, pv, emean, erstd, *enc_args(ee, 3))

    nt32 = node_type.astype(jnp.int32)
    vel = velocity.astype(F32)
    nmean, nrstd = node_stats(nt32, vel)
    nf, t_tab = node_enc(nt32, vel, nmean, nrstd, *enc_args(ne, 11),
                         ew[0][0], ew[0][1])

    for t in range(steps):
        ewa, ewb, ewc, eb1, ew2, eb2, ew3, eb3, elg, elb = ew[t]
        hu, hv = gather2(t_tab, su_idx, rv_idx)
        ef1n, ef1, ef2n, ef2 = edge_step(hu, hv, ef1, ef2, ewc, eb1, ew2,
                                         eb2, ew3, eb3, elg, elb)
        agg = scatter(ef1n, ef2n, r1_idx, r2_idx)
        nwa, nwb, nb1, nw2, nb2, nw3, nb3, nlg, nlb = nw[t]
        tn = min(t + 1, steps - 1)
        nf, t_tab = node_step(nf, agg[:n], agg[n_sc:n_sc + n], nwa, nwb, nb1,
                              nw2, nb2, nw3, nb3, nlg, nlb, ew[tn][0],
                              ew[tn][1])

    (w1, b1), (w2, b2), (w3, b3) = dec
    w3p = jnp.pad(w3.astype(F32), ((0, 0), (0, LAT - w3.shape[1])))
    b3p = jnp.pad(b3.astype(F32), (0, LAT - b3.shape[0]))
    out = decode(nf, w1.astype(F32), _vec(b1), w2.astype(F32), _vec(b2),
                 w3p, _vec(b3p))
    return out[:, :w3.shape[1]]


# speedup vs baseline: 2.7211x; 1.1364x over previous
import functools

import jax
import jax.numpy as jnp
from jax import lax
from jax.experimental import pallas as pl
from jax.experimental.pallas import tpu as pltpu
from jax.experimental.pallas import tpu_sc as plsc

F32 = jnp.float32
LAT = 64
NC = 2
NS = 16
NW = NC * NS
CH = 128

_SC_PARAMS = pltpu.CompilerParams(use_tc_tiling_on_sc=False)


def _dot(x, w):
    return lax.dot_general(x, w, (((1,), (0,)), ((), ())),
                           preferred_element_type=F32)


def _mm(x, w):
    return _dot(x, w)


def _ln(h, g, b):
    m = jnp.mean(h, axis=-1, keepdims=True)
    v = jnp.mean((h - m) * (h - m), axis=-1, keepdims=True)
    return (h - m) / jnp.sqrt(v + 1e-5) * g + b


@functools.lru_cache(maxsize=None)
def _make_gather2(pw, chunks, h_pad, d):
    mesh = plsc.VectorSubcoreMesh(core_axis_name="c", subcore_axis_name="s",
                                  num_cores=NC, num_subcores=NS)

    def body(tab, iu, iv, ou, ov, iu_v, iv_v, bu0, bu1, bv0, bv1,
             gsu0, gsu1, gsv0, gsv1, ssu0, ssu1, ssv0, ssv1):
        wid = lax.axis_index("s") * NC + lax.axis_index("c")
        pltpu.sync_copy(iu.at[wid], iu_v)
        pltpu.sync_copy(iv.at[wid], iv_v)
        base = wid * pw
        bus, bvs = (bu0, bu1), (bv0, bv1)
        gsu, gsv = (gsu0, gsu1), (gsv0, gsv1)
        ssu, ssv = (ssu0, ssu1), (ssv0, ssv1)

        gu = [None] * chunks
        gv = [None] * chunks
        su = [None] * chunks
        sv = [None] * chunks
        for j in range(chunks):
            b = j % 2
            if j >= 2:
                su[j - 2].wait()
                sv[j - 2].wait()
            off = j * CH
            gu[j] = pltpu.async_copy(tab.at[iu_v.at[pl.ds(off, CH)]],
                                     bus[b], gsu[b])
            gv[j] = pltpu.async_copy(tab.at[iv_v.at[pl.ds(off, CH)]],
                                     bvs[b], gsv[b])
            if j >= 1:
                p = j - 1
                pb = p % 2
                gu[p].wait()
                gv[p].wait()
                su[p] = pltpu.async_copy(bus[pb],
                                         ou.at[pl.ds(base + p * CH, CH)],
                                         ssu[pb])
                sv[p] = pltpu.async_copy(bvs[pb],
                                         ov.at[pl.ds(base + p * CH, CH)],
                                         ssv[pb])
        j = chunks - 1
        b = j % 2
        gu[j].wait()
        gv[j].wait()
        su[j] = pltpu.async_copy(bus[b], ou.at[pl.ds(base + j * CH, CH)],
                                 ssu[b])
        sv[j] = pltpu.async_copy(bvs[b], ov.at[pl.ds(base + j * CH, CH)],
                                 ssv[b])
        su[j - 1].wait()
        sv[j - 1].wait()
        su[j].wait()
        sv[j].wait()

    return pl.kernel(
        body,
        out_type=[jax.ShapeDtypeStruct((h_pad, d), F32)] * 2,
        mesh=mesh,
        compiler_params=_SC_PARAMS,
        scratch_types=[
            pltpu.VMEM((pw,), jnp.int32),
            pltpu.VMEM((pw,), jnp.int32),
            pltpu.VMEM((CH, d), F32),
            pltpu.VMEM((CH, d), F32),
            pltpu.VMEM((CH, d), F32),
            pltpu.VMEM((CH, d), F32),
        ] + [pltpu.SemaphoreType.DMA] * 8,
    )


@functools.lru_cache(maxsize=None)
def _make_scatter_add(n_sc, pw, chunks):
    mesh = plsc.VectorSubcoreMesh(core_axis_name="c", subcore_axis_name="s",
                                  num_cores=NC, num_subcores=NS)
    rs = n_sc // NS
    zb_rows = 128
    assert rs % zb_rows == 0

    def body(vals1, vals2, ridx1, ridx2, agg, idx1_v, idx2_v, idxc0, idxc1,
             lb0, lb1, zb, spmem, lsem0, lsem1, csem0, csem1, wsem):
        cid = lax.axis_index("c")
        sid = lax.axis_index("s")
        wid = sid * NC + cid
        idxcs = (idxc0, idxc1)
        lbs = (lb0, lb1)
        lsems = (lsem0, lsem1)
        csems = (csem0, csem1)

        def zrow(i, c):
            def zcol(k, c2):
                zb[i, pl.ds(k * 16, 16)] = jnp.zeros((16,), F32)
                return c2
            lax.fori_loop(0, LAT // 16, zcol, c)
            return c
        lax.fori_loop(0, zb_rows, zrow, 0)

        zd = [pltpu.async_copy(
            zb, spmem.at[pl.ds(sid * rs + k * zb_rows, zb_rows)], wsem)
            for k in range(rs // zb_rows)]
        for d in zd:
            d.wait()
        plsc.subcore_barrier()

        base = wid * pw
        pltpu.sync_copy(ridx1.at[wid], idx1_v)
        pltpu.sync_copy(ridx2.at[wid], idx2_v)

        def stage_idx(src, j, b):
            def cp(k, c):
                idxcs[b][pl.ds(k * 16, 16)] = src[pl.ds(j * CH + k * 16, 16)]
                return c
            lax.fori_loop(0, CH // 16, cp, 0)

        def src_of(s):
            if s < chunks:
                return vals1.at[pl.ds(base + s * CH, CH)], idx1_v, s
            return vals2.at[pl.ds(base + (s - chunks) * CH, CH)], idx2_v, s - chunks

        S = 2 * chunks
        ld = [None] * S
        sc = [None] * S
        for s in range(S):
            b = s % 2
            if s >= 2:
                sc[s - 2].wait()
            vref, isrc, jj = src_of(s)
            ld[s] = pltpu.async_copy(vref, lbs[b], lsems[b])
            stage_idx(isrc, jj, b)
            if s >= 1:
                p = s - 1
                pb = p % 2
                ld[p].wait()
                sc[p] = pltpu.async_copy(lbs[pb], spmem.at[idxcs[pb]],
                                         csems[pb], add=True)
        s = S - 1
        ld[s].wait()
        sc[s] = pltpu.async_copy(lbs[s % 2], spmem.at[idxcs[s % 2]],
                                 csems[s % 2], add=True)
        sc[s - 1].wait()
        sc[s].wait()
        plsc.subcore_barrier()

        wd = [pltpu.async_copy(
            spmem.at[pl.ds(sid * rs + k * zb_rows, zb_rows)],
            agg.at[pl.ds(cid * n_sc + sid * rs + k * zb_rows, zb_rows)],
            wsem) for k in range(rs // zb_rows)]
        for d in wd:
            d.wait()

    return pl.kernel(
        body,
        out_type=jax.ShapeDtypeStruct((NC * n_sc, LAT), F32),
        mesh=mesh,
        compiler_params=_SC_PARAMS,
        scratch_types=[
            pltpu.VMEM((pw,), jnp.int32),
            pltpu.VMEM((pw,), jnp.int32),
            pltpu.VMEM((CH,), jnp.int32),
            pltpu.VMEM((CH,), jnp.int32),
            pltpu.VMEM((CH, LAT), F32),
            pltpu.VMEM((CH, LAT), F32),
            pltpu.VMEM((zb_rows, LAT), F32),
            pltpu.VMEM_SHARED((n_sc, LAT), F32),
        ] + [pltpu.SemaphoreType.DMA] * 5,
    )


def _row_spec(rb, w):
    return pl.BlockSpec((rb, w), lambda i: (i, 0))


def _full_spec(shape):
    return pl.BlockSpec(shape, lambda i: tuple(0 for _ in shape))


def _edge_step_body(hu, hv, ef1, ef2, w1, b1, w2, b2, w3, b3, lg, lb,
                    e1n, e1o, e2n, e2o):
    u = hu[...]
    v = hv[...]

    def half(ef, a, bq):
        h = _mm(jnp.concatenate([a, bq, ef], axis=1), w1[...]) + b1[...]
        h = jnp.maximum(h, 0.0)
        h = jnp.maximum(_mm(h, w2[...]) + b2[...], 0.0)
        h = _mm(h, w3[...]) + b3[...]
        y = _ln(h, lg[...], lb[...])
        return y, ef + y

    e1 = ef1[...]
    e2 = ef2[...]
    y1, o1 = half(e1, u, v)
    y2, o2 = half(e2, v, u)
    e1n[...] = y1
    e1o[...] = o1
    e2n[...] = y2
    e2o[...] = o2


@functools.lru_cache(maxsize=None)
def _make_edge_step(h_pad, rb):
    grid = h_pad // rb
    row = _row_spec(rb, LAT)
    mat = _full_spec((LAT, LAT))
    vec = _full_spec((1, LAT))
    return pl.pallas_call(
        _edge_step_body,
        grid=(grid,),
        in_specs=[row, row, row, row, _full_spec((3 * LAT, LAT)), vec, mat,
                  vec, mat, vec, vec, vec],
        out_specs=[row, row, row, row],
        out_shape=[jax.ShapeDtypeStruct((h_pad, LAT), F32)] * 4,
    )


def _node_step_body(nf, a0, a1, w1, b1, w2, b2, w3, b3, lg, lb, nfo):
    agg = a0[...] + a1[...]
    h = _mm(jnp.concatenate([nf[...], agg], axis=1), w1[...]) + b1[...]
    h = jnp.maximum(h, 0.0)
    h = jnp.maximum(_mm(h, w2[...]) + b2[...], 0.0)
    h = _mm(h, w3[...]) + b3[...]
    nfo[...] = nf[...] + _ln(h, lg[...], lb[...])


@functools.lru_cache(maxsize=None)
def _make_node_step(n, rb):
    grid = n // rb
    row = _row_spec(rb, LAT)
    mat = _full_spec((LAT, LAT))
    vec = _full_spec((1, LAT))
    return pl.pallas_call(
        _node_step_body,
        grid=(grid,),
        in_specs=[row, row, row, _full_spec((2 * LAT, LAT)), vec, mat, vec,
                  mat, vec, vec, vec],
        out_specs=row,
        out_shape=jax.ShapeDtypeStruct((n, LAT), F32),
    )


def _node_feat(nt, vel):
    rb = vel.shape[0]
    col = lax.broadcasted_iota(jnp.int32, (rb, 16), 1)
    f = jnp.where(col == 0, vel[:, 0:1], 0.0)
    f = f + jnp.where(col == 1, vel[:, 1:2], 0.0)
    f = f + ((col - 2) == nt).astype(F32)
    return f


def _edge_feat(pu, pv, sign):
    rb = pu.shape[0]
    col = lax.broadcasted_iota(jnp.int32, (rb, 16), 1)
    rx = (pu[:, 0:1] - pv[:, 0:1]) * sign
    ry = (pu[:, 1:2] - pv[:, 1:2]) * sign
    nrm = jnp.sqrt(rx * rx + ry * ry)
    f = jnp.where(col == 0, rx, 0.0)
    f = f + jnp.where(col == 1, ry, 0.0)
    f = f + jnp.where(col == 2, nrm, 0.0)
    return f


@functools.lru_cache(maxsize=None)
def _make_node_stats(n, rb):
    grid = n // rb

    def body(nt, vel, mo, ro, ssum, ssq):
        i = pl.program_id(0)

        @pl.when(i == 0)
        def _():
            ssum[...] = jnp.zeros_like(ssum)
            ssq[...] = jnp.zeros_like(ssq)

        feat = _node_feat(nt[...], vel[...])
        ssum[...] += jnp.sum(feat, axis=0, keepdims=True)
        ssq[...] += jnp.sum(feat * feat, axis=0, keepdims=True)

        @pl.when(i == grid - 1)
        def _():
            mean = ssum[...] / float(n)
            var = jnp.maximum(ssq[...] / float(n) - mean * mean, 0.0)
            std = jnp.maximum(jnp.sqrt(var), 1e-8)
            mo[...] = mean
            ro[...] = 1.0 / std

    return pl.pallas_call(
        body,
        grid=(grid,),
        in_specs=[_row_spec(rb, 1), _row_spec(rb, 2)],
        out_specs=[_full_spec((1, 16))] * 2,
        out_shape=[jax.ShapeDtypeStruct((1, 16), F32)] * 2,
        scratch_shapes=[pltpu.VMEM((1, 16), F32), pltpu.VMEM((1, 16), F32)],
    )


@functools.lru_cache(maxsize=None)
def _make_edge_stats(h_pad, h, rb):
    grid = h_pad // rb

    def body(pu, pv, mo, ro, ssum, ssq):
        i = pl.program_id(0)

        @pl.when(i == 0)
        def _():
            ssum[...] = jnp.zeros_like(ssum)
            ssq[...] = jnp.zeros_like(ssq)

        f1 = _edge_feat(pu[...], pv[...], 1.0)
        f2 = _edge_feat(pu[...], pv[...], -1.0)
        rowid = i * rb + lax.broadcasted_iota(jnp.int32, (rb, 16), 0)
        valid = rowid < h
        f1 = jnp.where(valid, f1, 0.0)
        f2 = jnp.where(valid, f2, 0.0)
        ssum[...] += jnp.sum(f1 + f2, axis=0, keepdims=True)
        ssq[...] += jnp.sum(f1 * f1 + f2 * f2, axis=0, keepdims=True)

        @pl.when(i == grid - 1)
        def _():
            cnt = float(2 * h)
            mean = ssum[...] / cnt
            var = jnp.maximum(ssq[...] / cnt - mean * mean, 0.0)
            std = jnp.maximum(jnp.sqrt(var), 1e-8)
            mo[...] = mean
            ro[...] = 1.0 / std

    return pl.pallas_call(
        body,
        grid=(grid,),
        in_specs=[_row_spec(rb, 16), _row_spec(rb, 16)],
        out_specs=[_full_spec((1, 16))] * 2,
        out_shape=[jax.ShapeDtypeStruct((1, 16), F32)] * 2,
        scratch_shapes=[pltpu.VMEM((1, 16), F32), pltpu.VMEM((1, 16), F32)],
    )


def _enc_tail(x, w1, b1, w2, b2, w3, b3, lg, lb):
    h = jnp.maximum(_mm(x, w1[...]) + b1[...], 0.0)
    h = jnp.maximum(_mm(h, w2[...]) + b2[...], 0.0)
    h = _mm(h, w3[...]) + b3[...]
    return _ln(h, lg[...], lb[...])


@functools.lru_cache(maxsize=None)
def _make_node_enc(n, rb):
    grid = n // rb

    def body(nt, vel, mean, rstd, w1, b1, w2, b2, w3, b3, lg, lb, nfo):
        x = (_node_feat(nt[...], vel[...]) - mean[...]) * rstd[...]
        nfo[...] = _enc_tail(x, w1, b1, w2, b2, w3, b3, lg, lb)

    mat = _full_spec((LAT, LAT))
    vec = _full_spec((1, LAT))
    return pl.pallas_call(
        body,
        grid=(grid,),
        in_specs=[_row_spec(rb, 1), _row_spec(rb, 2),
                  _full_spec((1, 16)), _full_spec((1, 16)),
                  _full_spec((16, LAT)), vec, mat, vec, mat, vec, vec, vec],
        out_specs=_row_spec(rb, LAT),
        out_shape=jax.ShapeDtypeStruct((n, LAT), F32),
    )


@functools.lru_cache(maxsize=None)
def _make_edge_enc(h_pad, rb):
    grid = h_pad // rb

    def body(pu, pv, mean, rstd, w1, b1, w2, b2, w3, b3, lg, lb, e1o, e2o):
        x1 = (_edge_feat(pu[...], pv[...], 1.0) - mean[...]) * rstd[...]
        e1o[...] = _enc_tail(x1, w1, b1, w2, b2, w3, b3, lg, lb)
        x2 = (_edge_feat(pu[...], pv[...], -1.0) - mean[...]) * rstd[...]
        e2o[...] = _enc_tail(x2, w1, b1, w2, b2, w3, b3, lg, lb)

    mat = _full_spec((LAT, LAT))
    vec = _full_spec((1, LAT))
    return pl.pallas_call(
        body,
        grid=(grid,),
        in_specs=[_row_spec(rb, 16), _row_spec(rb, 16),
                  _full_spec((1, 16)), _full_spec((1, 16)),
                  _full_spec((16, LAT)), vec, mat, vec, mat, vec, vec, vec],
        out_specs=[_row_spec(rb, LAT)] * 2,
        out_shape=[jax.ShapeDtypeStruct((h_pad, LAT), F32)] * 2,
    )


@functools.lru_cache(maxsize=None)
def _make_decode(n, rb):
    grid = n // rb

    def body(nf, w1, b1, w2, b2, w3, b3, out):
        h = jnp.maximum(_mm(nf[...], w1[...]) + b1[...], 0.0)
        h = jnp.maximum(_mm(h, w2[...]) + b2[...], 0.0)
        out[...] = _mm(h, w3[...]) + b3[...]

    mat = _full_spec((LAT, LAT))
    vec = _full_spec((1, LAT))
    return pl.pallas_call(
        body,
        grid=(grid,),
        in_specs=[_row_spec(rb, LAT), mat, vec, mat, vec, mat, vec],
        out_specs=_row_spec(rb, LAT),
        out_shape=jax.ShapeDtypeStruct((n, LAT), F32),
    )


def _vec(b):
    return b.reshape(1, -1).astype(F32)


def kernel(node_type, velocity, cells, mesh_pos, is_trainning, params):
    n = velocity.shape[0]
    c = cells.shape[0]
    h = 3 * c

    cells = cells.astype(jnp.int32)
    c0, c1, c2 = cells[:, 0], cells[:, 1], cells[:, 2]
    su = jnp.concatenate([c0, c1, c2])
    rv = jnp.concatenate([c1, c2, c0])

    chunks = -(-h // (NW * CH))
    pw = chunks * CH
    h_pad = NW * pw
    pad = h_pad - h

    n_sc = -(-n // (NS * 128)) * (NS * 128)
    if n_sc == n:
        n_sc += NS * 128
    dummy = n_sc - 1

    zpad = jnp.zeros((pad,), jnp.int32)
    dpad = jnp.full((pad,), dummy, jnp.int32)
    su_idx = jnp.concatenate([su, zpad]).reshape(NW, pw)
    rv_idx = jnp.concatenate([rv, zpad]).reshape(NW, pw)
    r1_idx = jnp.concatenate([rv, dpad]).reshape(NW, pw)
    r2_idx = jnp.concatenate([su, dpad]).reshape(NW, pw)

    mp16 = jnp.pad(mesh_pos.astype(F32), ((0, 0), (0, 14)))

    gather16 = _make_gather2(pw, chunks, h_pad, 16)
    gather64 = _make_gather2(pw, chunks, h_pad, LAT)
    scatter = _make_scatter_add(n_sc, pw, chunks)

    rb_e = 512
    rb_n = 400
    assert h_pad % rb_e == 0 and n % rb_n == 0

    edge_stats = _make_edge_stats(h_pad, h, rb_e)
    edge_enc = _make_edge_enc(h_pad, rb_e)
    node_stats = _make_node_stats(n, rb_n)
    node_enc = _make_node_enc(n, rb_n)
    edge_step = _make_edge_step(h_pad, rb_e)
    node_step = _make_node_step(n, rb_n)
    decode = _make_decode(n, rb_n)

    ne = params["node_enc"]
    ee = params["edge_enc"]
    blocks = params["blocks"]
    dec = params["dec"]["mlp"]

    def enc_args(p, in_w):
        (w1, b1), (w2, b2), (w3, b3) = p["mlp"]
        g, b = p["ln"]
        w1p = jnp.pad(w1.astype(F32), ((0, 16 - in_w), (0, 0)))
        return (w1p, _vec(b1), w2.astype(F32), _vec(b2), w3.astype(F32),
                _vec(b3), _vec(g), _vec(b))

    ew = []
    nw = []
    for blk in blocks:
        (w1, b1), (w2, b2), (w3, b3) = blk["edge"]["mlp"]
        g, b = blk["edge"]["ln"]
        ew.append((w1.astype(F32), _vec(b1), w2.astype(F32), _vec(b2),
                   w3.astype(F32), _vec(b3), _vec(g), _vec(b)))
        (w1, b1), (w2, b2), (w3, b3) = blk["node"]["mlp"]
        g, b = blk["node"]["ln"]
        nw.append((w1.astype(F32), _vec(b1), w2.astype(F32), _vec(b2),
                   w3.astype(F32), _vec(b3), _vec(g), _vec(b)))

    steps = len(blocks)

    pu, pv = gather16(mp16, su_idx, rv_idx)
    emean, erstd = edge_stats(pu, pv)
    ef1, ef2 = edge_enc(pu, pv, emean, erstd, *enc_args(ee, 3))

    nt32 = node_type.astype(jnp.int32)
    vel = velocity.astype(F32)
    nmean, nrstd = node_stats(nt32, vel)
    nf = node_enc(nt32, vel, nmean, nrstd, *enc_args(ne, 11))

    for t in range(steps):
        hu, hv = gather64(nf, su_idx, rv_idx)
        ef1n, ef1, ef2n, ef2 = edge_step(hu, hv, ef1, ef2, *ew[t])
        agg = scatter(ef1n, ef2n, r1_idx, r2_idx)
        nf = node_step(nf, agg[:n], agg[n_sc:n_sc + n], *nw[t])

    (w1, b1), (w2, b2), (w3, b3) = dec
    w3p = jnp.pad(w3.astype(F32), ((0, 0), (0, LAT - w3.shape[1])))
    b3p = jnp.pad(b3.astype(F32), (0, LAT - b3.shape[0]))
    out = decode(nf, w1.astype(F32), _vec(b1), w2.astype(F32), _vec(b2),
                 w3p, _vec(b3p))
    return out[:, :w3.shape[1]]
